# Initial kernel scaffold; baseline (speedup 1.0000x reference)
import jax
import jax.numpy as jnp
from jax.experimental import pallas as pl


def kernel(inputs):
    raise NotImplementedError("write your pallas kernel here")


# speedup vs baseline: 16.3613x; 16.3613x over previous
import jax
import jax.numpy as jnp
import numpy as np
from jax import lax
from jax.experimental import pallas as pl
from jax.experimental.pallas import tpu as pltpu
from jax.experimental.pallas import tpu_sc as plsc

B = 32
T = 256
V = 1024
BEAM = 100
BLANK = V - 1
EPS = 1e-7
NEG = -1e30
LANES = 128
BIGI = 2**30

_PAIRS = [(k, j) for k in range(BEAM) for j in range(BEAM // (k + 1))]
NCAND = len(_PAIRS)
CPAD = 512
_K_OF = np.zeros((CPAD,), np.int32)
_J_OF = np.zeros((CPAD,), np.int32)
for _c, (_k, _j) in enumerate(_PAIRS):
    _K_OF[_c] = _k
    _J_OF[_c] = _j
_EK = np.zeros((LANES, CPAD), np.float32)
_EJ = np.zeros((LANES, CPAD), np.float32)
for _c, (_k, _j) in enumerate(_PAIRS):
    _EK[_k, _c] = 1.0
    _EJ[_j, _c] = 1.0


def _k1_body(x_ref, lv_ref, li_ref):
    lp = x_ref[0]
    pos = lax.broadcasted_iota(jnp.int32, (32, V), 1)
    lane = lax.broadcasted_iota(jnp.int32, (32, LANES), 1)
    accv0 = jnp.full((32, LANES), NEG, jnp.float32)
    acci0 = jnp.zeros((32, LANES), jnp.int32)

    def step(r, carry):
        cur, accv, acci = carry
        m = jnp.max(cur, axis=1, keepdims=True)
        hit = cur == m
        psel = jnp.min(jnp.where(hit, pos, BIGI), axis=1, keepdims=True)
        accv = jnp.where(lane == r, m, accv)
        acci = jnp.where(lane == r, psel, acci)
        cur = jnp.where(pos == psel, NEG, cur)
        return cur, accv, acci

    _, accv, acci = lax.fori_loop(0, BEAM, step, (lp, accv0, acci0))
    lv_ref[0] = accv
    li_ref[0] = acci


def _k2_body(lv_ref, ek_ref, ej_ref, pos_ref, sc_ref, s_ref):
    t = pl.program_id(0)
    lane = lax.broadcasted_iota(jnp.int32, (B, LANES), 1)

    @pl.when(t == 0)
    def _():
        s_ref[...] = jnp.where(lane == 0, 0.0, NEG).astype(jnp.float32)

    s = s_ref[...]
    l = lv_ref[:, 0, 0, :]
    cand = (jnp.dot(s, ek_ref[...], precision=lax.Precision.HIGHEST,
                    preferred_element_type=jnp.float32)
            + jnp.dot(l, ej_ref[...], precision=lax.Precision.HIGHEST,
                      preferred_element_type=jnp.float32))
    cpos = lax.broadcasted_iota(jnp.int32, (B, CPAD), 1)
    cand = jnp.where(cpos < NCAND, cand, NEG)

    accv0 = jnp.full((B, LANES), NEG, jnp.float32)
    accp0 = jnp.zeros((B, LANES), jnp.int32)

    def step(r, carry):
        cur, accv, accp = carry
        m = jnp.max(cur, axis=1, keepdims=True)
        hit = cur == m
        psel = jnp.min(jnp.where(hit, cpos, BIGI), axis=1, keepdims=True)
        accv = jnp.where(lane == r, m, accv)
        accp = jnp.where(lane == r, psel, accp)
        cur = jnp.where(cpos == psel, NEG, cur)
        return cur, accv, accp

    _, accv, accp = lax.fori_loop(0, BEAM, step, (cand, accv0, accp0))
    s_ref[...] = accv
    pos_ref[:, 0, 0, :] = accp
    sc_ref[...] = accv


def _k3_body(pos_hbm, li_hbm, kof_hbm, jof_hbm, out_hbm,
             pos_v, li_v, kof_v, jof_v, seq_v, out_v):
    b = lax.axis_index("s") * 2 + lax.axis_index("c")
    pltpu.sync_copy(pos_hbm.at[b], pos_v)
    pltpu.sync_copy(li_hbm.at[b], li_v)
    pltpu.sync_copy(kof_hbm, kof_v)
    pltpu.sync_copy(jof_hbm, jof_v)
    lanes = lax.iota(jnp.int32, 16)
    lane0 = lanes == 0

    def bt(i, ptr):
        tv = jnp.full((16,), T - 1 - i, jnp.int32)
        p = plsc.load_gather(pos_v, [tv * LANES + ptr])
        k = plsc.load_gather(kof_v, [p])
        j = plsc.load_gather(jof_v, [p])
        v = plsc.load_gather(li_v, [tv * LANES + j])
        plsc.store_scatter(seq_v, [tv], v, mask=lane0)
        return k

    lax.fori_loop(0, T, bt, jnp.zeros((16,), jnp.int32))

    def initc(i, c):
        plsc.store_scatter(out_v, [i * 16 + lanes],
                           jnp.full((16,), -1, jnp.int32))
        return c

    lax.fori_loop(0, T // 16, initc, jnp.int32(0))

    def col(i, base):
        iv = i * 16 + lanes
        sv = plsc.load_gather(seq_v, [iv])
        pv = plsc.load_gather(seq_v, [jnp.maximum(iv - 1, 0)])
        pv = jnp.where(iv == 0, jnp.int32(-1), pv)
        keep = (sv != pv) & (sv != BLANK)
        dest = base + plsc.cumsum(keep.astype(jnp.int32)) - 1
        dest = jnp.maximum(dest, 0)
        plsc.store_scatter(out_v, [dest], sv, mask=keep)
        return base + plsc.all_reduce_population_count(keep)

    lax.fori_loop(0, T // 16, col, jnp.zeros((16,), jnp.int32))
    pltpu.sync_copy(out_v, out_hbm.at[b])


def kernel(inputs):
    x = jnp.log(inputs + EPS)
    lv, li = pl.pallas_call(
        _k1_body,
        grid=(B, T // 32),
        in_specs=[pl.BlockSpec((1, 32, V), lambda b, tb: (b, tb, 0))],
        out_specs=[pl.BlockSpec((1, 32, LANES), lambda b, tb: (b, tb, 0)),
                   pl.BlockSpec((1, 32, LANES), lambda b, tb: (b, tb, 0))],
        out_shape=[jax.ShapeDtypeStruct((B, T, LANES), jnp.float32),
                   jax.ShapeDtypeStruct((B, T, LANES), jnp.int32)],
    )(x)

    pos4, scores = pl.pallas_call(
        _k2_body,
        grid=(T,),
        in_specs=[pl.BlockSpec((B, 1, 1, LANES), lambda t: (0, t, 0, 0)),
                  pl.BlockSpec((LANES, CPAD), lambda t: (0, 0)),
                  pl.BlockSpec((LANES, CPAD), lambda t: (0, 0))],
        out_specs=[pl.BlockSpec((B, 1, 1, LANES), lambda t: (0, t, 0, 0)),
                   pl.BlockSpec((B, LANES), lambda t: (0, 0))],
        out_shape=[jax.ShapeDtypeStruct((B, T, 1, LANES), jnp.int32),
                   jax.ShapeDtypeStruct((B, LANES), jnp.float32)],
        scratch_shapes=[pltpu.VMEM((B, LANES), jnp.float32)],
    )(lv.reshape(B, T, 1, LANES), jnp.asarray(_EK), jnp.asarray(_EJ))
    pos = pos4.reshape(B, T, LANES)

    mesh = plsc.VectorSubcoreMesh(core_axis_name="c", subcore_axis_name="s")
    decoded = pl.kernel(
        _k3_body,
        mesh=mesh,
        compiler_params=pltpu.CompilerParams(needs_layout_passes=False),
        out_type=jax.ShapeDtypeStruct((B, T), jnp.int32),
        scratch_types=[
            pltpu.VMEM((T * LANES,), jnp.int32),
            pltpu.VMEM((T * LANES,), jnp.int32),
            pltpu.VMEM((CPAD,), jnp.int32),
            pltpu.VMEM((CPAD,), jnp.int32),
            pltpu.VMEM((T,), jnp.int32),
            pltpu.VMEM((T,), jnp.int32),
        ],
    )(pos.reshape(B, T * LANES), li.reshape(B, T * LANES),
      jnp.asarray(_K_OF), jnp.asarray(_J_OF))

    return decoded, scores[:, :1]
